# baseline (speedup 1.0000x reference)
import functools

import jax
import jax.numpy as jnp
from jax import lax
from jax.experimental import pallas as pl

N = 10000
E = 160000
D_IN = 256
D_OUT = 768
NC = 3
NP = 10240
RB = 1024


def _lin_body(x_ref, w_ref, b_ref, y_ref):
    x = x_ref[...]
    w = w_ref[...]
    y = lax.dot_general(x, w, (((1,), (1,)), ((), ())),
                        preferred_element_type=jnp.float32)
    y_ref[0] = jnp.maximum(y + b_ref[0, 0, :], 0.0)


def _msg_linear(x_pad, W_lin, b3):
    grid = (NP // RB, NC)
    return pl.pallas_call(
        _lin_body,
        grid=grid,
        in_specs=[
            pl.BlockSpec((RB, D_IN), lambda i, c: (i, 0)),
            pl.BlockSpec((D_IN, D_IN), lambda i, c: (c, 0)),
            pl.BlockSpec((1, 1, D_IN), lambda i, c: (c, 0, 0)),
        ],
        out_specs=pl.BlockSpec((1, RB, D_IN), lambda i, c: (c, i, 0)),
        out_shape=jax.ShapeDtypeStruct((NC, NP, D_IN), jnp.float32),
    )(x_pad, W_lin, b3)


def _update_body(a_ref, x_ref, wua_ref, wux_ref, o_ref):
    x = x_ref[...]
    h = lax.dot_general(x, wux_ref[...], (((1,), (1,)), ((), ())),
                        preferred_element_type=jnp.float32)
    for c in range(NC):
        h = h + lax.dot_general(a_ref[c], wua_ref[c], (((1,), (1,)), ((), ())),
                                preferred_element_type=jnp.float32)
    h = jnp.maximum(h, 0.0)
    m = jnp.max(h, axis=1, keepdims=True)
    s = jnp.sum(jnp.exp(h - m), axis=1, keepdims=True)
    o_ref[...] = h - m - jnp.log(s)


def _update(aggr3, x_pad, Wua3, Wux):
    grid = (NP // RB,)
    return pl.pallas_call(
        _update_body,
        grid=grid,
        in_specs=[
            pl.BlockSpec((NC, RB, D_IN), lambda i: (0, i, 0)),
            pl.BlockSpec((RB, D_IN), lambda i: (i, 0)),
            pl.BlockSpec((NC, D_IN, D_IN), lambda i: (0, 0, 0)),
            pl.BlockSpec((D_IN, D_IN), lambda i: (0, 0)),
        ],
        out_specs=pl.BlockSpec((RB, D_IN), lambda i: (i, 0)),
        out_shape=jax.ShapeDtypeStruct((NP, D_IN), jnp.float32),
    )(aggr3, x_pad, Wua3, Wux)


def _segment_max_scaffold(Y3, src, dst):
    msg = jnp.take(Y3, src, axis=1)
    seg = jax.ops.segment_max(msg.transpose(1, 0, 2), dst, num_segments=NP)
    return jnp.maximum(jnp.where(jnp.isfinite(seg), seg, 0.0).transpose(1, 0, 2), Y3)


def kernel(x, edge_index, W_lin, b_lin, W_up):
    src, dst = edge_index[0], edge_index[1]
    x_pad = jnp.pad(x, ((0, NP - N), (0, 0)))
    b3 = b_lin.reshape(NC, 1, D_IN)
    Wua3 = W_up[:, :D_OUT].reshape(D_IN, NC, D_IN).transpose(1, 0, 2)
    Wux = W_up[:, D_OUT:]

    Y3 = _msg_linear(x_pad, W_lin, b3)
    aggr3 = _segment_max_scaffold(Y3, src, dst)
    out = _update(aggr3, x_pad, Wua3, Wux)
    return out[:N]


# speedup vs baseline: 2.7720x; 2.7720x over previous
import functools

import jax
import jax.numpy as jnp
from jax import lax
from jax.experimental import pallas as pl
from jax.experimental.pallas import tpu as pltpu
from jax.experimental.pallas import tpu_sc as plsc

N = 10000
E = 160000
D_IN = 256
D_OUT = 768
NC = 3
NW = 32
NPW = 320
NP = NW * NPW
RB = 1024
EB = 2000
G = 64
NGB = ((EB + G - 1) // G) * G + 16


def _lin_body(x_ref, w_ref, b_ref, y0_ref, y1_ref, y2_ref):
    y = lax.dot_general(x_ref[...], w_ref[...], (((1,), (1,)), ((), ())),
                        preferred_element_type=jnp.float32)
    y = jnp.maximum(y + b_ref[0, :], 0.0)
    y0_ref[...] = y[:, 0:D_IN]
    y1_ref[...] = y[:, D_IN:2 * D_IN]
    y2_ref[...] = y[:, 2 * D_IN:3 * D_IN]


def _msg_linear(x_pad, W_lin, b2):
    return pl.pallas_call(
        _lin_body,
        grid=(NP // RB,),
        in_specs=[
            pl.BlockSpec((RB, D_IN), lambda i: (i, 0)),
            pl.BlockSpec((D_OUT, D_IN), lambda i: (0, 0)),
            pl.BlockSpec((1, D_OUT), lambda i: (0, 0)),
        ],
        out_specs=[pl.BlockSpec((RB, D_IN), lambda i: (i, 0))] * NC,
        out_shape=[jax.ShapeDtypeStruct((NP, D_IN), jnp.float32)] * NC,
    )(x_pad, W_lin, b2)


def _segmax_body(y0, y1, y2, src_hbm, dst_hbm,
                 o0, o1, o2,
                 acc, srcb, dstb, cls, cld, rb, sem):
    wid = lax.axis_index("s") * 2 + lax.axis_index("c")
    base = wid * NPW

    def zero_body(i, _):
        cls[pl.ds(i * 16, 16)] = jnp.zeros((16,), jnp.int32)
        return 0
    lax.fori_loop(0, NGB // 16, zero_body, 0)

    for c, (yc, oc) in enumerate(((y0, o0), (y1, o1), (y2, o2))):
        pltpu.sync_copy(yc.at[pl.ds(base, NPW)], acc)

        def batch_body(b, _, yc=yc):
            off = b * EB
            pltpu.sync_copy(src_hbm.at[pl.ds(off, EB)], srcb)
            pltpu.sync_copy(dst_hbm.at[pl.ds(off, EB)], dstb)

            def filt_body(i, cnt):
                d = dstb[pl.ds(i * 16, 16)]
                s = srcb[pl.ds(i * 16, 16)]
                m = (d >= base) & (d < base + NPW)
                mi = jnp.where(m, 1, 0).astype(jnp.int32)
                incl = plsc.cumsum(mi)
                pos = cnt + incl - mi
                plsc.store_scatter(cls, [pos], s, mask=m)
                plsc.store_scatter(cld, [pos], d - base, mask=m)
                return cnt + plsc.all_reduce_population_count(m)[0]
            cnt = lax.fori_loop(0, EB // 16, filt_body, jnp.int32(0))

            def wave_body(w, _, yc=yc):
                woff = w * G
                pltpu.async_copy(yc.at[cls.at[pl.ds(woff, G)]], rb, sem).wait()
                nproc = jnp.minimum(cnt - woff, G)

                def edge_body(j, _):
                    ld = cld[pl.ds(woff + j, 16)][0]
                    for g in range(D_IN // 16):
                        sl = pl.ds(g * 16, 16)
                        acc[ld, sl] = jnp.maximum(acc[ld, sl], rb[j, sl])
                    return 0
                lax.fori_loop(0, nproc, edge_body, 0)
                return 0
            lax.fori_loop(0, (cnt + G - 1) // G, wave_body, 0)
            return 0
        lax.fori_loop(0, E // EB, batch_body, 0)

        pltpu.sync_copy(acc, oc.at[pl.ds(base, NPW)])


def _segment_max(Y3, src, dst):
    mesh = plsc.VectorSubcoreMesh(core_axis_name="c", subcore_axis_name="s")
    f = functools.partial(
        pl.kernel,
        out_type=[jax.ShapeDtypeStruct((NP, D_IN), jnp.float32)] * NC,
        mesh=mesh,
        compiler_params=pltpu.CompilerParams(use_tc_tiling_on_sc=False,
                                             needs_layout_passes=False),
        scratch_types=[
            pltpu.VMEM((NPW, D_IN), jnp.float32),
            pltpu.VMEM((EB,), jnp.int32),
            pltpu.VMEM((EB,), jnp.int32),
            pltpu.VMEM((NGB,), jnp.int32),
            pltpu.VMEM((NGB,), jnp.int32),
            pltpu.VMEM((G, D_IN), jnp.float32),
            pltpu.SemaphoreType.DMA,
        ],
    )(_segmax_body)
    return f(Y3[0], Y3[1], Y3[2], src, dst)


def _update_body(a0_ref, a1_ref, a2_ref, x_ref, wua_ref, wux_ref, o_ref):
    h = lax.dot_general(x_ref[...], wux_ref[...], (((1,), (1,)), ((), ())),
                        preferred_element_type=jnp.float32)
    for c, a_ref in enumerate((a0_ref, a1_ref, a2_ref)):
        h = h + lax.dot_general(a_ref[...], wua_ref[c],
                                (((1,), (1,)), ((), ())),
                                preferred_element_type=jnp.float32)
    h = jnp.maximum(h, 0.0)
    m = jnp.max(h, axis=1, keepdims=True)
    s = jnp.sum(jnp.exp(h - m), axis=1, keepdims=True)
    o_ref[...] = h - m - jnp.log(s)


def _update(aggr3, x_pad, Wua3, Wux):
    return pl.pallas_call(
        _update_body,
        grid=(NP // RB,),
        in_specs=[
            pl.BlockSpec((RB, D_IN), lambda i: (i, 0)),
            pl.BlockSpec((RB, D_IN), lambda i: (i, 0)),
            pl.BlockSpec((RB, D_IN), lambda i: (i, 0)),
            pl.BlockSpec((RB, D_IN), lambda i: (i, 0)),
            pl.BlockSpec((NC, D_IN, D_IN), lambda i: (0, 0, 0)),
            pl.BlockSpec((D_IN, D_IN), lambda i: (0, 0)),
        ],
        out_specs=pl.BlockSpec((RB, D_IN), lambda i: (i, 0)),
        out_shape=jax.ShapeDtypeStruct((NP, D_IN), jnp.float32),
    )(aggr3[0], aggr3[1], aggr3[2], x_pad, Wua3, Wux)


def kernel(x, edge_index, W_lin, b_lin, W_up):
    src, dst = edge_index[0], edge_index[1]
    x_pad = jnp.pad(x, ((0, NP - N), (0, 0)))
    b2 = b_lin.reshape(1, D_OUT)
    Wua3 = W_up[:, :D_OUT].reshape(D_IN, NC, D_IN).transpose(1, 0, 2)
    Wux = W_up[:, D_OUT:]

    Y3 = _msg_linear(x_pad, W_lin, b2)
    aggr3 = _segment_max(Y3, src, dst)
    out = _update(aggr3, x_pad, Wua3, Wux)
    return out[:N]
